# Initial kernel scaffold; baseline (speedup 1.0000x reference)
import jax
import jax.numpy as jnp
from jax.experimental import pallas as pl


def kernel(feat, segment_ids):
    raise NotImplementedError("write your pallas kernel here")


# speedup vs baseline: 3.6412x; 3.6412x over previous
import functools

import jax
import jax.numpy as jnp
from jax import lax
from jax.experimental import pallas as pl
from jax.experimental.pallas import tpu as pltpu
from jax.experimental.pallas import tpu_sc as plsc

N_ROWS = 100000
D = 128
SEGS = 256
NC, NS, L = 2, 16, 16
NW = NC * NS
BLOCK = 125
NBLOCKS = N_ROWS // BLOCK
BLOCKS_PER_W = NBLOCKS // NW
SEG_PER_TILE = SEGS // NS
QFULL = BLOCK // L
QREM = BLOCK - QFULL * L


def _sc_partials(feat3d, ids2d):
    mesh = plsc.VectorSubcoreMesh(
        core_axis_name="c", subcore_axis_name="s", num_cores=NC, num_subcores=NS
    )

    @functools.partial(
        pl.kernel,
        out_type=(
            jax.ShapeDtypeStruct((NC, SEGS, D), jnp.float32),
            jax.ShapeDtypeStruct((NC, NS, SEGS // L, L), jnp.float32),
        ),
        mesh=mesh,
        compiler_params=pltpu.CompilerParams(needs_layout_passes=False),
        scratch_types=[
            pltpu.VMEM((BLOCK,), jnp.int32),
            pltpu.VMEM((BLOCK, D), jnp.float32),
            pltpu.VMEM((SEG_PER_TILE, D), jnp.float32),
            pltpu.VMEM((SEGS // L, L), jnp.float32),
            pltpu.VMEM_SHARED((SEGS, D), jnp.float32),
        ],
    )
    def k(feat_hbm, ids_hbm, psum_hbm, pcnt_hbm,
          idx_v, rows_v, zrow_v, cnt_v, acc_sh):
        cid = lax.axis_index("c")
        sid = lax.axis_index("s")
        wid = sid * NC + cid

        one16 = jnp.full((L,), 1.0, dtype=jnp.float32)
        zero16 = jnp.zeros((L,), dtype=jnp.float32)
        for r in range(SEG_PER_TILE):
            for q in range(D // L):
                zrow_v[r, pl.ds(q * L, L)] = zero16
        for q in range(SEGS // L):
            cnt_v[q, :] = zero16

        pltpu.sync_copy(zrow_v, acc_sh.at[pl.ds(sid * SEG_PER_TILE, SEG_PER_TILE)])
        plsc.subcore_barrier()

        rem_mask = lax.iota(jnp.int32, L) >= (L - QREM)
        for j in range(BLOCKS_PER_W):
            jg = wid * BLOCKS_PER_W + j
            pltpu.sync_copy(ids_hbm.at[jg], idx_v)
            pltpu.sync_copy(feat_hbm.at[jg], rows_v)
            pltpu.sync_copy(rows_v, acc_sh.at[idx_v], add=True)
            for q in range(QFULL):
                iv = idx_v[pl.ds(q * L, L)]
                plsc.addupdate_scatter(
                    cnt_v, [lax.shift_right_logical(iv, 4), lax.bitwise_and(iv, 15)],
                    one16)
            iv = idx_v[pl.ds(BLOCK - L, L)]
            plsc.addupdate_scatter(
                cnt_v, [lax.shift_right_logical(iv, 4), lax.bitwise_and(iv, 15)],
                one16, mask=rem_mask)

        plsc.subcore_barrier()

        s0 = sid * SEG_PER_TILE
        pltpu.sync_copy(acc_sh.at[pl.ds(s0, SEG_PER_TILE)],
                        psum_hbm.at[cid, pl.ds(s0, SEG_PER_TILE)])
        pltpu.sync_copy(cnt_v, pcnt_hbm.at[cid, sid])

    return k(feat3d, ids2d)


def _combine_body(psum_ref, pcnt_ref, out_ref):
    s = psum_ref[0] + psum_ref[1]
    c = jnp.sum(pcnt_ref[...], axis=0)
    out_ref[...] = s / jnp.maximum(c, 1.0)[:, None]


def kernel(feat, segment_ids):
    ids2d = segment_ids.astype(jnp.int32).reshape(NBLOCKS, BLOCK)
    feat3d = feat.reshape(NBLOCKS, BLOCK, D)
    psum, pcnt = _sc_partials(feat3d, ids2d)
    pcnt2d = pcnt.reshape(NW, SEGS)
    return pl.pallas_call(
        _combine_body,
        out_shape=jax.ShapeDtypeStruct((SEGS, D), jnp.float32),
    )(psum, pcnt2d)


# speedup vs baseline: 4.2858x; 1.1770x over previous
import functools

import jax
import jax.numpy as jnp
from jax import lax
from jax.experimental import pallas as pl
from jax.experimental.pallas import tpu as pltpu
from jax.experimental.pallas import tpu_sc as plsc

N_ROWS = 100000
D = 128
SEGS = 256
NC, NS, L = 2, 16, 16
NW = NC * NS
BLOCK = 125
NBLOCKS = N_ROWS // BLOCK
BLOCKS_PER_W = NBLOCKS // NW
SEG_PER_TILE = SEGS // NS
QFULL = BLOCK // L
QREM = BLOCK - QFULL * L


def _sc_partials(feat3d, ids3d):
    mesh = plsc.VectorSubcoreMesh(
        core_axis_name="c", subcore_axis_name="s", num_cores=NC, num_subcores=NS
    )

    @functools.partial(
        pl.kernel,
        out_type=(
            jax.ShapeDtypeStruct((NC, SEGS, D), jnp.float32),
            jax.ShapeDtypeStruct((NC, NS, SEGS // L, L), jnp.float32),
        ),
        mesh=mesh,
        compiler_params=pltpu.CompilerParams(needs_layout_passes=False),
        scratch_types=[
            pltpu.VMEM((BLOCKS_PER_W, BLOCK), jnp.int32),
            pltpu.VMEM((SEG_PER_TILE, D), jnp.float32),
            pltpu.VMEM((SEGS // L, L), jnp.float32),
            pltpu.VMEM_SHARED((SEGS, D), jnp.float32),
            [pltpu.VMEM((BLOCK, D), jnp.float32) for _ in range(3)],
            [pltpu.SemaphoreType.DMA for _ in range(3)],
            [pltpu.SemaphoreType.DMA for _ in range(3)],
        ],
    )
    def k(feat_hbm, ids_hbm, psum_hbm, pcnt_hbm,
          idx_all, zrow_v, cnt_v, acc_sh, rows_bufs, lsems, ssems):
        cid = lax.axis_index("c")
        sid = lax.axis_index("s")
        wid = sid * NC + cid

        pltpu.sync_copy(ids_hbm.at[wid], idx_all)

        one16 = jnp.full((L,), 1.0, dtype=jnp.float32)
        zero16 = jnp.zeros((L,), dtype=jnp.float32)
        for r in range(SEG_PER_TILE):
            for q in range(D // L):
                zrow_v[r, pl.ds(q * L, L)] = zero16
        for q in range(SEGS // L):
            cnt_v[q, :] = zero16

        pltpu.sync_copy(zrow_v, acc_sh.at[pl.ds(sid * SEG_PER_TILE, SEG_PER_TILE)])
        plsc.subcore_barrier()

        rem_mask = lax.iota(jnp.int32, L) >= (L - QREM)
        NBUF = 3
        load_d = [None] * NBUF
        scat_d = [None] * NBUF
        for b in range(min(NBUF - 1, BLOCKS_PER_W)):
            load_d[b] = pltpu.async_copy(
                feat_hbm.at[wid * BLOCKS_PER_W + b], rows_bufs[b], lsems[b])
        for j in range(BLOCKS_PER_W):
            cur = j % NBUF
            load_d[cur].wait()
            scat_d[cur] = pltpu.async_copy(
                rows_bufs[cur], acc_sh.at[idx_all.at[j]], ssems[cur], add=True)
            jn = j + NBUF - 1
            if jn < BLOCKS_PER_W:
                nxt = jn % NBUF
                if scat_d[nxt] is not None:
                    scat_d[nxt].wait()
                load_d[nxt] = pltpu.async_copy(
                    feat_hbm.at[wid * BLOCKS_PER_W + jn], rows_bufs[nxt], lsems[nxt])
            for q in range(QFULL):
                iv = idx_all[j, pl.ds(q * L, L)]
                plsc.addupdate_scatter(
                    cnt_v, [lax.shift_right_logical(iv, 4), lax.bitwise_and(iv, 15)],
                    one16)
            iv = idx_all[j, pl.ds(BLOCK - L, L)]
            plsc.addupdate_scatter(
                cnt_v, [lax.shift_right_logical(iv, 4), lax.bitwise_and(iv, 15)],
                one16, mask=rem_mask)

        for b in range(NBUF):
            if scat_d[b] is not None:
                scat_d[b].wait()
        plsc.subcore_barrier()

        s0 = sid * SEG_PER_TILE
        pltpu.sync_copy(acc_sh.at[pl.ds(s0, SEG_PER_TILE)],
                        psum_hbm.at[cid, pl.ds(s0, SEG_PER_TILE)])
        pltpu.sync_copy(cnt_v, pcnt_hbm.at[cid, sid])

    return k(feat3d, ids3d)


def _combine_body(psum_ref, pcnt_ref, out_ref):
    s = psum_ref[0] + psum_ref[1]
    c = jnp.sum(pcnt_ref[...], axis=0)
    out_ref[...] = s / jnp.maximum(c, 1.0)[:, None]


def kernel(feat, segment_ids):
    ids3d = segment_ids.astype(jnp.int32).reshape(NW, BLOCKS_PER_W, BLOCK)
    feat3d = feat.reshape(NBLOCKS, BLOCK, D)
    psum, pcnt = _sc_partials(feat3d, ids3d)
    pcnt2d = pcnt.reshape(NW, SEGS)
    return pl.pallas_call(
        _combine_body,
        out_shape=jax.ShapeDtypeStruct((SEGS, D), jnp.float32),
    )(psum, pcnt2d)


# speedup vs baseline: 8.0075x; 1.8684x over previous
import functools

import jax
import jax.numpy as jnp
from jax import lax
from jax.experimental import pallas as pl
from jax.experimental.pallas import tpu as pltpu
from jax.experimental.pallas import tpu_sc as plsc

N_ROWS = 100000
D = 128
SEGS = 256
NC, NS, L = 2, 16, 16
NW = NC * NS
G = 160
NG = N_ROWS // G
NSLOT = -(-NG // NW)
GA, GB = 128, 32
SEG_PER_TILE = SEGS // NS
NBUF = 3


def _sc_partials(featg, ids1d):
    mesh = plsc.VectorSubcoreMesh(
        core_axis_name="c", subcore_axis_name="s", num_cores=NC, num_subcores=NS
    )

    @functools.partial(
        pl.kernel,
        out_type=(
            jax.ShapeDtypeStruct((NC, SEGS, D), jnp.float32),
            jax.ShapeDtypeStruct((NC, NS, SEGS // L, L), jnp.float32),
        ),
        mesh=mesh,
        compiler_params=pltpu.CompilerParams(needs_layout_passes=False),
        scratch_types=[
            pltpu.VMEM((SEG_PER_TILE, D), jnp.float32),
            pltpu.VMEM((SEGS // L + 1, L), jnp.float32),
            pltpu.VMEM_SHARED((SEGS + 1, D), jnp.float32),
            [pltpu.VMEM((G, D), jnp.float32) for _ in range(NBUF)],
            [pltpu.VMEM((GA,), jnp.int32) for _ in range(NBUF)],
            [pltpu.VMEM((GB,), jnp.int32) for _ in range(NBUF)],
            [pltpu.SemaphoreType.DMA for _ in range(NBUF)],
            [pltpu.SemaphoreType.DMA for _ in range(NBUF)],
        ],
    )
    def k(feat_hbm, ids_hbm, psum_hbm, pcnt_hbm,
          zrow_v, cnt_v, acc_sh, rows_bufs, idxa_bufs, idxb_bufs, lsems, ssems):
        cid = lax.axis_index("c")
        sid = lax.axis_index("s")
        wid = sid * NC + cid

        one16 = jnp.full((L,), 1.0, dtype=jnp.float32)
        zero16 = jnp.zeros((L,), dtype=jnp.float32)
        trash16 = jnp.full((L,), SEGS, dtype=jnp.int32)
        for r in range(SEG_PER_TILE):
            for q in range(D // L):
                zrow_v[r, pl.ds(q * L, L)] = zero16
        for q in range(SEGS // L + 1):
            cnt_v[q, :] = zero16

        pltpu.sync_copy(zrow_v, acc_sh.at[pl.ds(sid * SEG_PER_TILE, SEG_PER_TILE)])
        plsc.subcore_barrier()

        def slot_group(t):
            jg = wid + NW * t
            valid = jg < NG
            return jnp.where(valid, jg, 0), valid

        def issue_loads(t, b):
            jg, _ = slot_group(t)
            la = pltpu.async_copy(feat_hbm.at[jg], rows_bufs[b], lsems[b])
            lb = pltpu.async_copy(ids_hbm.at[pl.ds(jg * G, GA)], idxa_bufs[b],
                                  lsems[b])
            lc = pltpu.async_copy(ids_hbm.at[pl.ds(jg * G + GA, GB)], idxb_bufs[b],
                                  lsems[b])
            return (la, lb, lc)

        load_d = [None] * NBUF
        scat_d = [None] * NBUF
        for b in range(NBUF - 1):
            load_d[b] = issue_loads(b, b)

        for t in range(NSLOT):
            cur = t % NBUF
            for d0 in load_d[cur]:
                d0.wait()
            _, valid = slot_group(t)
            for q in range(G // L):
                if q * L < GA:
                    ref, off = idxa_bufs[cur], q * L
                else:
                    ref, off = idxb_bufs[cur], q * L - GA
                iv = ref[pl.ds(off, L)]
                iv = jnp.where(valid, iv, trash16)
                ref[pl.ds(off, L)] = iv
                plsc.addupdate_scatter(
                    cnt_v, [lax.shift_right_logical(iv, 4), lax.bitwise_and(iv, 15)],
                    one16)
            scat_d[cur] = (
                pltpu.async_copy(rows_bufs[cur].at[pl.ds(0, GA)],
                                 acc_sh.at[idxa_bufs[cur]], ssems[cur], add=True),
                pltpu.async_copy(rows_bufs[cur].at[pl.ds(GA, GB)],
                                 acc_sh.at[idxb_bufs[cur]], ssems[cur], add=True),
            )
            jn = t + NBUF - 1
            if jn < NSLOT:
                nxt = jn % NBUF
                if scat_d[nxt] is not None:
                    for d0 in scat_d[nxt]:
                        d0.wait()
                    scat_d[nxt] = None
                load_d[nxt] = issue_loads(jn, nxt)

        for b in range(NBUF):
            if scat_d[b] is not None:
                for d0 in scat_d[b]:
                    d0.wait()
        plsc.subcore_barrier()

        s0 = sid * SEG_PER_TILE
        pltpu.sync_copy(acc_sh.at[pl.ds(s0, SEG_PER_TILE)],
                        psum_hbm.at[cid, pl.ds(s0, SEG_PER_TILE)])
        pltpu.sync_copy(cnt_v.at[pl.ds(0, SEGS // L)], pcnt_hbm.at[cid, sid])

    return k(featg, ids1d)


def _combine_body(psum_ref, pcnt_ref, out_ref):
    s = psum_ref[0] + psum_ref[1]
    c = jnp.sum(pcnt_ref[...], axis=0)
    out_ref[...] = s / jnp.maximum(c, 1.0)[:, None]


def kernel(feat, segment_ids):
    featg = feat.reshape(NG, G, D)
    ids1d = segment_ids.astype(jnp.int32)
    psum, pcnt = _sc_partials(featg, ids1d)
    pcnt2d = pcnt.reshape(NW, SEGS)
    return pl.pallas_call(
        _combine_body,
        out_shape=jax.ShapeDtypeStruct((SEGS, D), jnp.float32),
    )(psum, pcnt2d)


# speedup vs baseline: 8.5353x; 1.0659x over previous
import functools

import jax
import jax.numpy as jnp
from jax import lax
from jax.experimental import pallas as pl
from jax.experimental.pallas import tpu as pltpu
from jax.experimental.pallas import tpu_sc as plsc

N_ROWS = 100000
D = 128
SEGS = 256
NC, NS, L = 2, 16, 16
NW = NC * NS
G = 160
NG = N_ROWS // G
NSLOT = -(-NG // NW)
GA, GB = 128, 32
SEG_PER_TILE = SEGS // NS
NBUF = 4


def _sc_partials(featg, ids1d):
    mesh = plsc.VectorSubcoreMesh(
        core_axis_name="c", subcore_axis_name="s", num_cores=NC, num_subcores=NS
    )

    @functools.partial(
        pl.kernel,
        out_type=(
            jax.ShapeDtypeStruct((NC, SEGS, D), jnp.float32),
            jax.ShapeDtypeStruct((NC, NS, SEGS // L, L), jnp.float32),
        ),
        mesh=mesh,
        compiler_params=pltpu.CompilerParams(needs_layout_passes=False),
        scratch_types=[
            pltpu.VMEM((SEG_PER_TILE, D), jnp.float32),
            pltpu.VMEM((SEGS // L + 1, L), jnp.float32),
            pltpu.VMEM_SHARED((SEGS + 1, D), jnp.float32),
            [pltpu.VMEM((G, D), jnp.float32) for _ in range(NBUF)],
            [pltpu.VMEM((GA,), jnp.int32) for _ in range(NBUF)],
            [pltpu.VMEM((GB,), jnp.int32) for _ in range(NBUF)],
            [pltpu.SemaphoreType.DMA for _ in range(NBUF)],
            [pltpu.SemaphoreType.DMA for _ in range(NBUF)],
        ],
    )
    def k(feat_hbm, ids_hbm, psum_hbm, pcnt_hbm,
          zrow_v, cnt_v, acc_sh, rows_bufs, idxa_bufs, idxb_bufs, lsems, ssems):
        cid = lax.axis_index("c")
        sid = lax.axis_index("s")
        wid = sid * NC + cid

        one16 = jnp.full((L,), 1.0, dtype=jnp.float32)
        zero16 = jnp.zeros((L,), dtype=jnp.float32)
        trash16 = jnp.full((L,), SEGS, dtype=jnp.int32)
        for r in range(SEG_PER_TILE):
            for q in range(D // L):
                zrow_v[r, pl.ds(q * L, L)] = zero16
        for q in range(SEGS // L + 1):
            cnt_v[q, :] = zero16

        pltpu.sync_copy(zrow_v, acc_sh.at[pl.ds(sid * SEG_PER_TILE, SEG_PER_TILE)])
        plsc.subcore_barrier()

        def slot_group(t):
            jg = wid + NW * t
            valid = jg < NG
            return jnp.where(valid, jg, 0), valid

        def issue_loads(t, b):
            jg, _ = slot_group(t)
            la = pltpu.async_copy(feat_hbm.at[jg], rows_bufs[b], lsems[b])
            lb = pltpu.async_copy(ids_hbm.at[pl.ds(jg * G, GA)], idxa_bufs[b],
                                  lsems[b])
            lc = pltpu.async_copy(ids_hbm.at[pl.ds(jg * G + GA, GB)], idxb_bufs[b],
                                  lsems[b])
            return (la, lb, lc)

        load_d = [None] * NBUF
        scat_d = [None] * NBUF
        for b in range(NBUF - 1):
            load_d[b] = issue_loads(b, b)

        for t in range(NSLOT):
            cur = t % NBUF
            for d0 in load_d[cur]:
                d0.wait()
            _, valid = slot_group(t)
            for q in range(G // L):
                if q * L < GA:
                    ref, off = idxa_bufs[cur], q * L
                else:
                    ref, off = idxb_bufs[cur], q * L - GA
                iv = ref[pl.ds(off, L)]
                iv = jnp.where(valid, iv, trash16)
                ref[pl.ds(off, L)] = iv
                plsc.addupdate_scatter(
                    cnt_v, [lax.shift_right_logical(iv, 4), lax.bitwise_and(iv, 15)],
                    one16)
            scat_d[cur] = (
                pltpu.async_copy(rows_bufs[cur].at[pl.ds(0, GA)],
                                 acc_sh.at[idxa_bufs[cur]], ssems[cur], add=True),
                pltpu.async_copy(rows_bufs[cur].at[pl.ds(GA, GB)],
                                 acc_sh.at[idxb_bufs[cur]], ssems[cur], add=True),
            )
            jn = t + NBUF - 1
            if jn < NSLOT:
                nxt = jn % NBUF
                if scat_d[nxt] is not None:
                    for d0 in scat_d[nxt]:
                        d0.wait()
                    scat_d[nxt] = None
                load_d[nxt] = issue_loads(jn, nxt)

        for b in range(NBUF):
            if scat_d[b] is not None:
                for d0 in scat_d[b]:
                    d0.wait()
        plsc.subcore_barrier()

        s0 = sid * SEG_PER_TILE
        pltpu.sync_copy(acc_sh.at[pl.ds(s0, SEG_PER_TILE)],
                        psum_hbm.at[cid, pl.ds(s0, SEG_PER_TILE)])
        pltpu.sync_copy(cnt_v.at[pl.ds(0, SEGS // L)], pcnt_hbm.at[cid, sid])

    return k(featg, ids1d)


def _combine_body(psum_ref, pcnt_ref, out_ref):
    s = psum_ref[0] + psum_ref[1]
    c = jnp.sum(pcnt_ref[...], axis=0)
    out_ref[...] = s / jnp.maximum(c, 1.0)[:, None]


def kernel(feat, segment_ids):
    featg = feat.reshape(NG, G, D)
    ids1d = segment_ids.astype(jnp.int32)
    psum, pcnt = _sc_partials(featg, ids1d)
    pcnt2d = pcnt.reshape(NW, SEGS)
    return pl.pallas_call(
        _combine_body,
        out_shape=jax.ShapeDtypeStruct((SEGS, D), jnp.float32),
    )(psum, pcnt2d)
